# baseline (speedup 1.0000x reference)
import functools

import jax
import jax.numpy as jnp
from jax import lax
from jax.experimental import pallas as pl
from jax.experimental.pallas import tpu as pltpu
from jax.experimental.pallas import tpu_sc as plsc

VOCAB = 16
D = 1024
TOKENS = 4 * 8192

_info = plsc.get_sparse_core_info()
NC, NS = _info.num_cores, _info.num_subcores
NW = NC * NS
TPW = TOKENS // NW
BURST = 8
DEPTH_B = 4


def _sc_body(ids_hbm, w_hbm, cm_hbm, out_hbm,
             ids_sm, cm_sm, ids_sh, cm_sh, trc_sh, wrow_v, prow_v, tab_v,
             sems, semw):
    sid = lax.axis_index("s")
    wid = sid * NC + lax.axis_index("c")
    base = wid * TPW

    cp_ids = pltpu.async_copy(ids_hbm.at[pl.ds(base, TPW)], ids_sh.at[sid],
                              sems.at[0])
    cp_cm = pltpu.async_copy(cm_hbm, cm_sh.at[sid], sems.at[1])
    cp_w = pltpu.async_copy(w_hbm, tab_v.at[pl.ds(0, VOCAB), pl.ds(0, D)],
                            sems.at[2])
    cp_cm.wait()
    pltpu.sync_copy(cm_sh.at[sid], cm_sm)

    pltpu.sync_copy(w_hbm.at[cm_sm[sid]], wrow_v)

    def rc_chunk(j, carry):
        v = wrow_v[pl.ds(D - 16 * j - 16, 16)]
        prow_v[pl.ds(16 * j, 16)] = lax.rev(v, (0,))
        return carry

    lax.fori_loop(0, D // 16, rc_chunk, 0)
    pltpu.sync_copy(prow_v, trc_sh.at[sid])
    cp_ids.wait()
    pltpu.sync_copy(ids_sh.at[sid], ids_sm)
    cp_w.wait()
    plsc.subcore_barrier()
    pltpu.sync_copy(trc_sh, tab_v.at[pl.ds(0, VOCAB), pl.ds(D, D)])

    def issue_burst(i, carry):
        t = i * BURST
        for u in range(BURST):
            pltpu.async_copy(tab_v.at[ids_sm[t + u]],
                             out_hbm.at[base + t + u], semw)
        return carry

    def wait_burst():
        pltpu.make_async_copy(tab_v.at[pl.ds(0, BURST)],
                              out_hbm.at[pl.ds(base, BURST)], semw).wait()

    def step(i, carry):
        issue_burst(i, carry)
        wait_burst()
        return carry

    def drain(j, carry):
        wait_burst()
        return carry

    lax.fori_loop(0, DEPTH_B, issue_burst, 0)
    lax.fori_loop(DEPTH_B, TPW // BURST, step, 0)
    lax.fori_loop(0, DEPTH_B, drain, 0)


def _sc_write(ids, weight, comp_map):
    mesh = plsc.VectorSubcoreMesh(core_axis_name="c", subcore_axis_name="s")
    f = functools.partial(
        pl.kernel,
        mesh=mesh,
        out_type=jax.ShapeDtypeStruct((TOKENS, 2 * D), jnp.float32),
        scratch_types=[
            pltpu.SMEM((TPW,), jnp.int32),
            pltpu.SMEM((VOCAB,), jnp.int32),
            pltpu.VMEM_SHARED((NS, TPW), jnp.int32),
            pltpu.VMEM_SHARED((NS, VOCAB), jnp.int32),
            pltpu.VMEM_SHARED((VOCAB, D), jnp.float32),
            pltpu.VMEM((D,), jnp.float32),
            pltpu.VMEM((D,), jnp.float32),
            pltpu.VMEM((VOCAB, 2 * D), jnp.float32),
            pltpu.SemaphoreType.DMA((3,)),
            pltpu.SemaphoreType.DMA,
        ],
    )(_sc_body)
    return f(ids, weight, comp_map)


def kernel(input_ids, weight, comp_map):
    ids = input_ids.reshape(-1)
    out = _sc_write(ids, weight, comp_map)
    return out.reshape(input_ids.shape[0], input_ids.shape[1], 2 * D)


# speedup vs baseline: 1.0134x; 1.0134x over previous
import functools

import jax
import jax.numpy as jnp
from jax import lax
from jax.experimental import pallas as pl
from jax.experimental.pallas import tpu as pltpu
from jax.experimental.pallas import tpu_sc as plsc

VOCAB = 16
D = 1024
TOKENS = 4 * 8192

_info = plsc.get_sparse_core_info()
NC, NS = _info.num_cores, _info.num_subcores
NW = NC * NS
TPW = TOKENS // NW
BURST = 8
DEPTH_B = 2


def _sc_body(ids_hbm, w_hbm, cm_hbm, out_hbm,
             ids_sm, cm_sm, ids_sh, cm_sh, trc_sh, wrow_v, prow_v, tab_v,
             sems, semw):
    sid = lax.axis_index("s")
    wid = sid * NC + lax.axis_index("c")
    base = wid * TPW

    cp_ids = pltpu.async_copy(ids_hbm.at[pl.ds(base, TPW)], ids_sh.at[sid],
                              sems.at[0])
    cp_cm = pltpu.async_copy(cm_hbm, cm_sh.at[sid], sems.at[1])
    cp_w = pltpu.async_copy(w_hbm, tab_v.at[pl.ds(0, VOCAB), pl.ds(0, D)],
                            sems.at[2])
    cp_cm.wait()
    pltpu.sync_copy(cm_sh.at[sid], cm_sm)

    pltpu.sync_copy(w_hbm.at[cm_sm[sid]], wrow_v)

    def rc_chunk(j, carry):
        v = wrow_v[pl.ds(D - 16 * j - 16, 16)]
        prow_v[pl.ds(16 * j, 16)] = lax.rev(v, (0,))
        return carry

    lax.fori_loop(0, D // 16, rc_chunk, 0)
    pltpu.sync_copy(prow_v, trc_sh.at[sid])
    cp_ids.wait()
    pltpu.sync_copy(ids_sh.at[sid], ids_sm)
    cp_w.wait()
    plsc.subcore_barrier()
    pltpu.sync_copy(trc_sh, tab_v.at[pl.ds(0, VOCAB), pl.ds(D, D)])

    def issue_burst(i, carry):
        t = i * BURST
        for u in range(BURST):
            pltpu.async_copy(tab_v.at[ids_sm[t + u]],
                             out_hbm.at[base + t + u], semw)
        return carry

    def wait_burst():
        pltpu.make_async_copy(tab_v.at[pl.ds(0, BURST)],
                              out_hbm.at[pl.ds(base, BURST)], semw).wait()

    def step(i, carry):
        issue_burst(i, carry)
        wait_burst()
        return carry

    def drain(j, carry):
        wait_burst()
        return carry

    lax.fori_loop(0, DEPTH_B, issue_burst, 0)
    lax.fori_loop(DEPTH_B, TPW // BURST, step, 0)
    lax.fori_loop(0, DEPTH_B, drain, 0)


def _sc_write(ids, weight, comp_map):
    mesh = plsc.VectorSubcoreMesh(core_axis_name="c", subcore_axis_name="s")
    f = functools.partial(
        pl.kernel,
        mesh=mesh,
        out_type=jax.ShapeDtypeStruct((TOKENS, 2 * D), jnp.float32),
        scratch_types=[
            pltpu.SMEM((TPW,), jnp.int32),
            pltpu.SMEM((VOCAB,), jnp.int32),
            pltpu.VMEM_SHARED((NS, TPW), jnp.int32),
            pltpu.VMEM_SHARED((NS, VOCAB), jnp.int32),
            pltpu.VMEM_SHARED((VOCAB, D), jnp.float32),
            pltpu.VMEM((D,), jnp.float32),
            pltpu.VMEM((D,), jnp.float32),
            pltpu.VMEM((VOCAB, 2 * D), jnp.float32),
            pltpu.SemaphoreType.DMA((3,)),
            pltpu.SemaphoreType.DMA,
        ],
    )(_sc_body)
    return f(ids, weight, comp_map)


def kernel(input_ids, weight, comp_map):
    ids = input_ids.reshape(-1)
    out = _sc_write(ids, weight, comp_map)
    return out.reshape(input_ids.shape[0], input_ids.shape[1], 2 * D)
